# baseline (speedup 1.0000x reference)
import functools

import jax
import jax.numpy as jnp
from jax import lax
from jax.experimental import pallas as pl
from jax.experimental.pallas import tpu as pltpu
from jax.experimental.pallas import tpu_sc as plsc

N = 16 * 512 * 512
K_TOP = N * 10 // 100
SHIFT = 19
NBINS = 0x44000000 >> SHIFT
LSTRIDE = NBINS + 1
ROWS = NBINS // 128
NW = 32
PER_TILE = N // NW
CHUNK = 8192
NCHUNK = PER_TILE // CHUNK
VPC = CHUNK // 16
SDIV = 16
NS = N // SDIV
PER_TILE_S = NS // NW
K_SAMPLE = K_TOP / SDIV


def _bce_body(p_ref, g_ref, o_ref):
    p = p_ref[...]
    g = g_ref[...]
    lp = jnp.maximum(jnp.log(p), -100.0)
    l1 = jnp.maximum(jnp.log(1.0 - p), -100.0)
    o_ref[...] = -(g * lp + (1.0 - g) * l1)


def _bce(p2, g2):
    return pl.pallas_call(
        _bce_body,
        grid=(8,),
        in_specs=[pl.BlockSpec((1024, 512), lambda i: (i, 0))] * 2,
        out_specs=pl.BlockSpec((1024, 512), lambda i: (i, 0)),
        out_shape=jax.ShapeDtypeStruct((8192, 512), jnp.float32),
    )(p2, g2)


def _bce_sample(p2, g2):
    return pl.pallas_call(
        _bce_body,
        grid=(4,),
        in_specs=[pl.BlockSpec((128, 512), lambda i: (i * 16, 0))] * 2,
        out_specs=pl.BlockSpec((128, 512), lambda i: (i, 0)),
        out_shape=jax.ShapeDtypeStruct((512, 512), jnp.float32),
    )(p2, g2)


def _hist_kernel_body(res_hbm, out_hbm, chunk_v, hist_v, out_v):
    wid = lax.axis_index("s") * 2 + lax.axis_index("c")

    zero16 = jnp.zeros((16,), jnp.int32)
    ones16 = jnp.ones((16,), jnp.int32)
    lane_base = lax.broadcasted_iota(jnp.int32, (16,), 0) * LSTRIDE

    def _zero(i, _):
        hist_v[pl.ds(i * 16, 16)] = zero16
        return 0

    lax.fori_loop(0, LSTRIDE * 16 // 16, _zero, 0, unroll=8)

    pltpu.sync_copy(res_hbm.at[pl.ds(wid * 16, 16)], chunk_v)

    for r in range(16):
        def _vreg(j, _, r=r):
            v = chunk_v[r, pl.ds(j * 16, 16)]
            bits = plsc.bitcast(v, jnp.int32)
            b = jnp.maximum(bits >> SHIFT, 0)
            b = jnp.minimum(b, NBINS - 1)
            plsc.addupdate_scatter(hist_v, [b + lane_base], ones16)
            return 0

        lax.fori_loop(0, 512 // 16, _vreg, 0, unroll=8)

    def _red(j, _):
        acc = zero16
        for l in range(16):
            acc = acc + hist_v[pl.ds(l * LSTRIDE + j * 16, 16)]
        out_v[pl.ds(j * 16, 16)] = acc
        return 0

    lax.fori_loop(0, NBINS // 16, _red, 0, unroll=4)

    pltpu.sync_copy(out_v, out_hbm.at[wid])


def _hist(res_flat):
    mesh = plsc.VectorSubcoreMesh(core_axis_name="c", subcore_axis_name="s")
    fn = pl.kernel(
        _hist_kernel_body,
        out_type=jax.ShapeDtypeStruct((NW, NBINS), jnp.int32),
        mesh=mesh,
        scratch_types=[
            pltpu.VMEM((16, 512), jnp.float32),
            pltpu.VMEM((LSTRIDE * 16,), jnp.int32),
            pltpu.VMEM((NBINS,), jnp.int32),
        ],
        compiler_params=pltpu.CompilerParams(needs_layout_passes=False),
    )
    return fn(res_flat)


def _sel_body(hist_ref, p_ref, g_ref, out_ref, acc):
    i = pl.program_id(0)

    @pl.when(i == 0)
    def _():
        h1 = jnp.sum(hist_ref[...].astype(jnp.float32), axis=0)
        h2 = h1.reshape(ROWS, 128)
        li = lax.broadcasted_iota(jnp.int32, (128, 128), 0)
        lj = lax.broadcasted_iota(jnp.int32, (128, 128), 1)
        upper = (li >= lj).astype(jnp.float32)
        s_in = jnp.dot(h2, upper, preferred_element_type=jnp.float32)
        rowtot = s_in[:, 0:1]
        ri = lax.broadcasted_iota(jnp.int32, (ROWS, ROWS), 0)
        rj = lax.broadcasted_iota(jnp.int32, (ROWS, ROWS), 1)
        below = (rj > ri).astype(jnp.float32)
        strict = jnp.dot(below, rowtot, preferred_element_type=jnp.float32)
        suffix = s_in + strict
        bidx = (lax.broadcasted_iota(jnp.int32, (ROWS, 128), 0) * 128
                + lax.broadcasted_iota(jnp.int32, (ROWS, 128), 1))
        cand = jnp.where(suffix >= float(K_SAMPLE), bidx, -1)
        bstar = jnp.max(cand)
        acc[0] = 0.0
        acc[1] = 0.0
        acc[2] = lax.bitcast_convert_type(bstar << SHIFT, jnp.float32)

    p = p_ref[...]
    g = g_ref[...]
    lp = jnp.maximum(jnp.log(p), -100.0)
    l1 = jnp.maximum(jnp.log(1.0 - p), -100.0)
    r = -(g * lp + (1.0 - g) * l1)
    t = acc[2]
    m = r >= t
    acc[0] += jnp.sum(jnp.where(m, r, 0.0))
    acc[1] += jnp.sum(m.astype(jnp.float32))

    @pl.when(i == 7)
    def _():
        kf = jnp.float32(K_TOP)
        val = (acc[0] + (kf - acc[1]) * acc[2]) / kf
        out_ref[...] = jnp.full((1, 1), val, jnp.float32)


def _select(hist2, p2, g2):
    return pl.pallas_call(
        _sel_body,
        grid=(8,),
        in_specs=[
            pl.BlockSpec((NW, NBINS), lambda i: (0, 0)),
            pl.BlockSpec((1024, 512), lambda i: (i, 0)),
            pl.BlockSpec((1024, 512), lambda i: (i, 0)),
        ],
        out_specs=pl.BlockSpec((1, 1), lambda i: (0, 0)),
        out_shape=jax.ShapeDtypeStruct((1, 1), jnp.float32),
        scratch_shapes=[pltpu.SMEM((4,), jnp.float32)],
    )(hist2, p2, g2)


def kernel(preds, gt_masks):
    p2 = preds.reshape(8192, 512)
    g2 = gt_masks.reshape(8192, 512)
    samp = _bce_sample(p2, g2)
    hist = _hist(samp)
    out = _select(hist, p2, g2)
    return out[0, 0]


# speedup vs baseline: 1.0393x; 1.0393x over previous
import functools

import jax
import jax.numpy as jnp
from jax import lax
from jax.experimental import pallas as pl
from jax.experimental.pallas import tpu as pltpu
from jax.experimental.pallas import tpu_sc as plsc

N = 16 * 512 * 512
K_TOP = N * 10 // 100
SHIFT = 19
NBINS = 0x44000000 >> SHIFT
LSTRIDE = NBINS + 1
ROWS = NBINS // 128
NW = 32
PER_TILE = N // NW
CHUNK = 8192
NCHUNK = PER_TILE // CHUNK
VPC = CHUNK // 16
SDIV = 16
NS = N // SDIV
PER_TILE_S = NS // NW
K_SAMPLE = K_TOP / SDIV


def _bce_body(p_ref, g_ref, o_ref):
    p = p_ref[...]
    g = g_ref[...]
    lp = jnp.maximum(jnp.log(p), -100.0)
    l1 = jnp.maximum(jnp.log(1.0 - p), -100.0)
    o_ref[...] = -(g * lp + (1.0 - g) * l1)


def _bce(p2, g2):
    return pl.pallas_call(
        _bce_body,
        grid=(8,),
        in_specs=[pl.BlockSpec((1024, 512), lambda i: (i, 0))] * 2,
        out_specs=pl.BlockSpec((1024, 512), lambda i: (i, 0)),
        out_shape=jax.ShapeDtypeStruct((8192, 512), jnp.float32),
    )(p2, g2)


def _bce_sample(p2, g2):
    return pl.pallas_call(
        _bce_body,
        grid=(4,),
        in_specs=[pl.BlockSpec((128, 512), lambda i: (i * 16, 0))] * 2,
        out_specs=pl.BlockSpec((128, 512), lambda i: (i, 0)),
        out_shape=jax.ShapeDtypeStruct((512, 512), jnp.float32),
    )(p2, g2)


def _hist_kernel_body(res_hbm, out_hbm, chunk_v, hist_v, out_v):
    wid = lax.axis_index("s") * 2 + lax.axis_index("c")

    zero16 = jnp.zeros((16,), jnp.int32)
    ones16 = jnp.ones((16,), jnp.int32)
    lane_base = lax.broadcasted_iota(jnp.int32, (16,), 0) * LSTRIDE

    def _zero(i, _):
        hist_v[pl.ds(i * 16, 16)] = zero16
        return 0

    lax.fori_loop(0, LSTRIDE * 16 // 16, _zero, 0, unroll=8)

    pltpu.sync_copy(res_hbm.at[pl.ds(wid * 16, 16)], chunk_v)

    for r in range(16):
        def _vreg(j, _, r=r):
            v = chunk_v[r, pl.ds(j * 16, 16)]
            bits = plsc.bitcast(v, jnp.int32)
            b = jnp.maximum(bits >> SHIFT, 0)
            b = jnp.minimum(b, NBINS - 1)
            plsc.addupdate_scatter(hist_v, [b + lane_base], ones16)
            return 0

        lax.fori_loop(0, 512 // 16, _vreg, 0, unroll=8)

    def _red(j, _):
        acc = zero16
        for l in range(16):
            acc = acc + hist_v[pl.ds(l * LSTRIDE + j * 16, 16)]
        out_v[pl.ds(j * 16, 16)] = acc
        return 0

    lax.fori_loop(0, NBINS // 16, _red, 0, unroll=4)

    pltpu.sync_copy(out_v, out_hbm.at[wid])


def _hist(res_flat):
    mesh = plsc.VectorSubcoreMesh(core_axis_name="c", subcore_axis_name="s")
    fn = pl.kernel(
        _hist_kernel_body,
        out_type=jax.ShapeDtypeStruct((NW, NBINS), jnp.int32),
        mesh=mesh,
        scratch_types=[
            pltpu.VMEM((16, 512), jnp.float32),
            pltpu.VMEM((LSTRIDE * 16,), jnp.int32),
            pltpu.VMEM((NBINS,), jnp.int32),
        ],
        compiler_params=pltpu.CompilerParams(needs_layout_passes=False),
    )
    return fn(res_flat)


def _sel_body(hist_ref, p_ref, g_ref, out_ref, acc):
    i = pl.program_id(0)

    @pl.when(i == 0)
    def _():
        h1 = jnp.sum(hist_ref[...].astype(jnp.float32), axis=0)
        h2 = h1.reshape(ROWS, 128)
        li = lax.broadcasted_iota(jnp.int32, (128, 128), 0)
        lj = lax.broadcasted_iota(jnp.int32, (128, 128), 1)
        upper = (li >= lj).astype(jnp.float32)
        s_in = jnp.dot(h2, upper, preferred_element_type=jnp.float32)
        rowtot = s_in[:, 0:1]
        ri = lax.broadcasted_iota(jnp.int32, (ROWS, ROWS), 0)
        rj = lax.broadcasted_iota(jnp.int32, (ROWS, ROWS), 1)
        below = (rj > ri).astype(jnp.float32)
        strict = jnp.dot(below, rowtot, preferred_element_type=jnp.float32)
        suffix = s_in + strict
        bidx = (lax.broadcasted_iota(jnp.int32, (ROWS, 128), 0) * 128
                + lax.broadcasted_iota(jnp.int32, (ROWS, 128), 1))
        cand = jnp.where(suffix >= float(K_SAMPLE), bidx, -1)
        bstar = jnp.max(cand)
        acc[0] = 0.0
        acc[1] = 0.0
        acc[2] = lax.bitcast_convert_type(bstar << SHIFT, jnp.float32)

    t2 = acc[2] * jnp.float32(1.4426950408889634)
    sacc = jnp.zeros((8, 512), jnp.float32)
    cacc = jnp.zeros((8, 512), jnp.float32)
    for jj in range(128):
        p = p_ref[pl.ds(jj * 8, 8), :]
        g = g_ref[pl.ds(jj * 8, 8), :]
        lp = jnp.maximum(jnp.log2(p), -144.26950408889634)
        l1 = jnp.maximum(jnp.log2(1.0 - p), -144.26950408889634)
        r2 = g * (l1 - lp) - l1
        m = r2 >= t2
        sacc = sacc + jnp.where(m, r2, 0.0)
        cacc = cacc + m.astype(jnp.float32)
    acc[0] += jnp.sum(sacc)
    acc[1] += jnp.sum(cacc)

    @pl.when(i == 7)
    def _():
        kf = jnp.float32(K_TOP)
        val = (jnp.float32(0.6931471805599453) * acc[0]
               + (kf - acc[1]) * acc[2]) / kf
        out_ref[...] = jnp.full((1, 1), val, jnp.float32)


def _select(hist2, p2, g2):
    return pl.pallas_call(
        _sel_body,
        grid=(8,),
        in_specs=[
            pl.BlockSpec((NW, NBINS), lambda i: (0, 0)),
            pl.BlockSpec((1024, 512), lambda i: (i, 0)),
            pl.BlockSpec((1024, 512), lambda i: (i, 0)),
        ],
        out_specs=pl.BlockSpec((1, 1), lambda i: (0, 0)),
        out_shape=jax.ShapeDtypeStruct((1, 1), jnp.float32),
        scratch_shapes=[pltpu.SMEM((4,), jnp.float32)],
    )(hist2, p2, g2)


def kernel(preds, gt_masks):
    p2 = preds.reshape(8192, 512)
    g2 = gt_masks.reshape(8192, 512)
    samp = _bce_sample(p2, g2)
    hist = _hist(samp)
    out = _select(hist, p2, g2)
    return out[0, 0]
